# baseline (speedup 1.0000x reference)
import functools

import jax
import jax.numpy as jnp
from jax import lax
from jax.experimental import pallas as pl
from jax.experimental.pallas import tpu as pltpu
from jax.experimental.pallas import tpu_sc as plsc

FT_OUT = 512
MAX_F = 32
N_FT = 49152
N_FFT = 768
B = 4096

NC = 2
NS = 16
NW = NC * NS
R = B // NW
NO = FT_OUT // 32

_PVEC = 3 * FT_OUT
_WG = 8


def _rne_bf16_bits(x):
    u = lax.bitcast_convert_type(x, jnp.uint32)
    return (u + 0x7FFF + ((u >> 16) & 1)) >> 16


def _pack_rows(rows):
    lo = _rne_bf16_bits(rows[..., : FT_OUT // 2])
    hi = _rne_bf16_bits(rows[..., FT_OUT // 2:])
    return lax.bitcast_convert_type(lo | (hi << 16), jnp.int32)


def _combine_body(ft_ref, fft_ref, ftb_ref, fftb_ref, outw_ref, o_ref, pv_ref):
    ft = ft_ref[...].reshape(_WG, N_FFT, FT_OUT)
    s = ft + fft_ref[...][None]
    o_ref[...] = _pack_rows(s).reshape(_WG * N_FFT, FT_OUT // 2)
    bias = ftb_ref[...] + fftb_ref[...]
    ow = outw_ref[...]
    rows = jnp.concatenate([bias, ow[:, :FT_OUT], ow[:, FT_OUT:]], axis=0)
    pv_ref[...] = _pack_rows(rows)


def _build_w2(ft_w, fft_w, ft_b, fft_b, out_w):
    return pl.pallas_call(
        _combine_body,
        grid=(N_FT // (_WG * N_FFT),),
        in_specs=[
            pl.BlockSpec((_WG * N_FFT, FT_OUT), lambda i: (i, 0)),
            pl.BlockSpec((N_FFT, FT_OUT), lambda i: (0, 0)),
            pl.BlockSpec((1, FT_OUT), lambda i: (0, 0)),
            pl.BlockSpec((1, FT_OUT), lambda i: (0, 0)),
            pl.BlockSpec((1, 2 * FT_OUT), lambda i: (0, 0)),
        ],
        out_specs=[
            pl.BlockSpec((_WG * N_FFT, FT_OUT // 2), lambda i: (i, 0)),
            pl.BlockSpec((3, FT_OUT // 2), lambda i: (0, 0)),
        ],
        out_shape=[
            jax.ShapeDtypeStruct((N_FT, FT_OUT // 2), jnp.int32),
            jax.ShapeDtypeStruct((3, FT_OUT // 2), jnp.int32),
        ],
    )(ft_w, fft_w, ft_b.reshape(1, FT_OUT), fft_b.reshape(1, FT_OUT), out_w)


def _sc_body(w2, vals_flat, stm, nstm, pvec, obvec, out_hbm,
             stm_v, nstm_v, vals_v, pvec_v, ob_v,
             g0, g1, tbuf, logit_v,
             sem0, sem1):
    wid = lax.axis_index("s") * NC + lax.axis_index("c")
    base = wid * R

    pltpu.sync_copy(stm.at[pl.ds(base * MAX_F, R * MAX_F)], stm_v)
    pltpu.sync_copy(nstm.at[pl.ds(base * MAX_F, R * MAX_F)], nstm_v)
    pltpu.sync_copy(vals_flat.at[pl.ds(base * MAX_F, R * MAX_F)], vals_v)
    pltpu.sync_copy(pvec, pvec_v)
    pltpu.sync_copy(obvec, ob_v)

    def issue(p, g, sem):
        sl = pl.ds(p * 2 * MAX_F, 2 * MAX_F)
        pltpu.async_copy(w2.at[stm_v.at[sl]], g.at[pl.ds(0, 2 * MAX_F)], sem)
        pltpu.async_copy(w2.at[nstm_v.at[sl]],
                         g.at[pl.ds(2 * MAX_F, 2 * MAX_F)], sem)

    def drain(g, sem):
        pltpu.make_async_copy(w2.at[pl.ds(0, 4 * MAX_F)], g, sem).wait()

    def compute(r, g, off):
        wv0 = vals_v[pl.ds(r * MAX_F, 16)]
        wv1 = vals_v[pl.ds(r * MAX_F + 16, 16)]
        ws = [wv0[f] for f in range(16)] + [wv1[f] for f in range(16)]

        def _splat_bf16(w):
            wv = jnp.broadcast_to(w, (16,))
            return plsc.pack(wv, wv, format=plsc.PackFormat.INTERLEAVED)

        wb = [_splat_bf16(w) for w in ws]
        zero = jnp.zeros((32,), jnp.bfloat16)
        one = jnp.full((32,), 1.0, jnp.bfloat16)

        def obody(o, t):
            o16 = o * 16
            sl = pl.ds(o16, 16)

            def bf(x):
                return plsc.bitcast(x, jnp.bfloat16)

            a_s = bf(pvec_v[sl])
            a_n = a_s
            for f in range(MAX_F):
                a_s = a_s + wb[f] * bf(g[off + f, sl])
                a_n = a_n + wb[f] * bf(g[off + 2 * MAX_F + f, sl])
            hs = jnp.minimum(jnp.maximum(a_s, zero), one)
            hn = jnp.minimum(jnp.maximum(a_n, zero), one)
            return (t + hs * bf(pvec_v[pl.ds(FT_OUT // 2 + o16, 16)])
                      + hn * bf(pvec_v[pl.ds(FT_OUT + o16, 16)]))

        t = lax.fori_loop(0, NO, obody, zero)
        ta, tb = plsc.unpack(t, format=plsc.PackFormat.INTERLEAVED)
        tbuf[pl.ds(r * 16, 16)] = ta + tb

    issue(0, g0, sem0)

    def body(i, carry):
        p0 = 2 * i
        r0 = 4 * i
        issue(p0 + 1, g1, sem1)
        drain(g0, sem0)
        compute(r0, g0, 0)
        compute(r0 + 1, g0, MAX_F)

        @pl.when(p0 + 2 < R // 2)
        def _():
            issue(p0 + 2, g0, sem0)

        drain(g1, sem1)
        compute(r0 + 2, g1, 0)
        compute(r0 + 3, g1, MAX_F)
        return carry

    lax.fori_loop(0, R // 4, body, 0)

    ob = ob_v[...]
    lanes16 = lax.iota(jnp.int32, 16) * 16
    for j in range(R // 16):
        s = ob
        for p in range(16):
            idx = lanes16 + (j * 256 + p)
            s = s + plsc.load_gather(tbuf, [idx])
        logit_v[pl.ds(j * 16, 16)] = 1.0 / (1.0 + jnp.exp(-s))
    pltpu.sync_copy(logit_v, out_hbm.at[pl.ds(base, R)])


@jax.jit
def _sc_bag(w2, vals_flat, stm, nstm, pvec, obvec):
    mesh = plsc.VectorSubcoreMesh(core_axis_name="c", subcore_axis_name="s")
    f = pl.kernel(
        _sc_body,
        mesh=mesh,
        out_type=jax.ShapeDtypeStruct((B,), jnp.float32),
        compiler_params=pltpu.CompilerParams(needs_layout_passes=False),
        scratch_types=[
            pltpu.VMEM((R * MAX_F,), jnp.int32),
            pltpu.VMEM((R * MAX_F,), jnp.int32),
            pltpu.VMEM((R * MAX_F,), jnp.float32),
            pltpu.VMEM((_PVEC // 2,), jnp.int32),
            pltpu.VMEM((16,), jnp.float32),
            pltpu.VMEM((4 * MAX_F, FT_OUT // 2), jnp.int32),
            pltpu.VMEM((4 * MAX_F, FT_OUT // 2), jnp.int32),
            pltpu.VMEM((R * 16,), jnp.float32),
            pltpu.VMEM((R,), jnp.float32),
            pltpu.SemaphoreType.DMA,
            pltpu.SemaphoreType.DMA,
        ],
    )
    return f(w2, vals_flat, stm, nstm, pvec, obvec)


def kernel(values, stm_indices, nstm_indices, ft_w, ft_b, fft_w, fft_b,
           out_w, out_b):
    w2, pv = _build_w2(ft_w, fft_w, ft_b, fft_b, out_w)
    obvec = jnp.broadcast_to(out_b, (16,)).astype(jnp.float32)
    out = _sc_bag(w2, values.reshape(-1),
                  stm_indices.astype(jnp.int32).reshape(-1),
                  nstm_indices.astype(jnp.int32).reshape(-1),
                  pv.reshape(_PVEC // 2), obvec)
    return out.reshape(B, 1)


# speedup vs baseline: 1.1119x; 1.1119x over previous
import functools

import jax
import jax.numpy as jnp
from jax import lax
from jax.experimental import pallas as pl
from jax.experimental.pallas import tpu as pltpu
from jax.experimental.pallas import tpu_sc as plsc

FT_OUT = 512
MAX_F = 32
N_FT = 49152
N_FFT = 768
B = 4096

NC = 2
NS = 16
NW = NC * NS
R = B // NW
NO = FT_OUT // 32

_PVEC = 3 * FT_OUT
_WG = 8


def _rne_bf16_bits(x):
    u = lax.bitcast_convert_type(x, jnp.uint32)
    return (u + 0x7FFF + ((u >> 16) & 1)) >> 16


def _pack_rows(rows):
    lo = _rne_bf16_bits(rows[..., : FT_OUT // 2])
    hi = _rne_bf16_bits(rows[..., FT_OUT // 2:])
    return lax.bitcast_convert_type(lo | (hi << 16), jnp.int32)


def _combine_body(ft_ref, fft_ref, ftb_ref, fftb_ref, outw_ref, o_ref, pv_ref):
    ft = ft_ref[...].reshape(_WG, N_FFT, FT_OUT)
    s = ft + fft_ref[...][None]
    o_ref[...] = _pack_rows(s).reshape(_WG * N_FFT, FT_OUT // 2)
    bias = ftb_ref[...] + fftb_ref[...]
    ow = outw_ref[...]
    rows = jnp.concatenate([bias, ow[:, :FT_OUT], ow[:, FT_OUT:]], axis=0)
    pv_ref[...] = _pack_rows(rows)


def _build_w2(ft_w, fft_w, ft_b, fft_b, out_w):
    return pl.pallas_call(
        _combine_body,
        grid=(N_FT // (_WG * N_FFT),),
        in_specs=[
            pl.BlockSpec((_WG * N_FFT, FT_OUT), lambda i: (i, 0)),
            pl.BlockSpec((N_FFT, FT_OUT), lambda i: (0, 0)),
            pl.BlockSpec((1, FT_OUT), lambda i: (0, 0)),
            pl.BlockSpec((1, FT_OUT), lambda i: (0, 0)),
            pl.BlockSpec((1, 2 * FT_OUT), lambda i: (0, 0)),
        ],
        out_specs=[
            pl.BlockSpec((_WG * N_FFT, FT_OUT // 2), lambda i: (i, 0)),
            pl.BlockSpec((3, FT_OUT // 2), lambda i: (0, 0)),
        ],
        out_shape=[
            jax.ShapeDtypeStruct((N_FT, FT_OUT // 2), jnp.int32),
            jax.ShapeDtypeStruct((3, FT_OUT // 2), jnp.int32),
        ],
    )(ft_w, fft_w, ft_b.reshape(1, FT_OUT), fft_b.reshape(1, FT_OUT), out_w)


def _sc_body(w2, vals_flat, stm, nstm, pvec, obvec, out_hbm,
             stm_v, nstm_v, vals_v, pvec_v, ob_v,
             g0, g1, g2, g3, tbuf, logit_v,
             sem0, sem1, sem2, sem3):
    wid = lax.axis_index("s") * NC + lax.axis_index("c")
    base = wid * R

    pltpu.sync_copy(stm.at[pl.ds(base * MAX_F, R * MAX_F)], stm_v)
    pltpu.sync_copy(nstm.at[pl.ds(base * MAX_F, R * MAX_F)], nstm_v)
    pltpu.sync_copy(vals_flat.at[pl.ds(base * MAX_F, R * MAX_F)], vals_v)
    pltpu.sync_copy(pvec, pvec_v)
    pltpu.sync_copy(obvec, ob_v)

    def issue(r, g, sem):
        sl = pl.ds(r * MAX_F, MAX_F)
        pltpu.async_copy(w2.at[stm_v.at[sl]], g.at[pl.ds(0, MAX_F)], sem)
        pltpu.async_copy(w2.at[nstm_v.at[sl]],
                         g.at[pl.ds(MAX_F, MAX_F)], sem)

    def drain(g, sem):
        pltpu.make_async_copy(w2.at[pl.ds(0, 2 * MAX_F)], g, sem).wait()

    def compute(r, g):
        wv0 = vals_v[pl.ds(r * MAX_F, 16)]
        wv1 = vals_v[pl.ds(r * MAX_F + 16, 16)]
        ws = [wv0[f] for f in range(16)] + [wv1[f] for f in range(16)]

        def _splat_bf16(w):
            wv = jnp.broadcast_to(w, (16,))
            return plsc.pack(wv, wv, format=plsc.PackFormat.INTERLEAVED)

        wb = [_splat_bf16(w) for w in ws]
        zero = jnp.zeros((32,), jnp.bfloat16)
        one = jnp.full((32,), 1.0, jnp.bfloat16)

        def obody(o, t):
            o16 = o * 16
            sl = pl.ds(o16, 16)

            def bf(x):
                return plsc.bitcast(x, jnp.bfloat16)

            a_s = bf(pvec_v[sl])
            a_n = a_s
            for f in range(MAX_F):
                a_s = a_s + wb[f] * bf(g[f, sl])
                a_n = a_n + wb[f] * bf(g[MAX_F + f, sl])
            hs = jnp.minimum(jnp.maximum(a_s, zero), one)
            hn = jnp.minimum(jnp.maximum(a_n, zero), one)
            return (t + hs * bf(pvec_v[pl.ds(FT_OUT // 2 + o16, 16)])
                      + hn * bf(pvec_v[pl.ds(FT_OUT + o16, 16)]))

        t = lax.fori_loop(0, NO, obody, zero)
        ta, tb = plsc.unpack(t, format=plsc.PackFormat.INTERLEAVED)
        tbuf[pl.ds(r * 16, 16)] = ta + tb

    slots = ((g0, sem0), (g1, sem1), (g2, sem2), (g3, sem3))
    for k in range(4):
        issue(k, *slots[k])

    def body(i, carry):
        r0 = 4 * i
        for k in range(4):
            r = r0 + k
            g, sem = slots[k]
            drain(g, sem)
            compute(r, g)

            @pl.when(r + 4 < R)
            def _():
                issue(r + 4, g, sem)
        return carry

    lax.fori_loop(0, R // 4, body, 0)

    ob = ob_v[...]
    lanes16 = lax.iota(jnp.int32, 16) * 16
    for j in range(R // 16):
        s = ob
        for p in range(16):
            idx = lanes16 + (j * 256 + p)
            s = s + plsc.load_gather(tbuf, [idx])
        logit_v[pl.ds(j * 16, 16)] = 1.0 / (1.0 + jnp.exp(-s))
    pltpu.sync_copy(logit_v, out_hbm.at[pl.ds(base, R)])


@jax.jit
def _sc_bag(w2, vals_flat, stm, nstm, pvec, obvec):
    mesh = plsc.VectorSubcoreMesh(core_axis_name="c", subcore_axis_name="s")
    f = pl.kernel(
        _sc_body,
        mesh=mesh,
        out_type=jax.ShapeDtypeStruct((B,), jnp.float32),
        compiler_params=pltpu.CompilerParams(needs_layout_passes=False),
        scratch_types=[
            pltpu.VMEM((R * MAX_F,), jnp.int32),
            pltpu.VMEM((R * MAX_F,), jnp.int32),
            pltpu.VMEM((R * MAX_F,), jnp.float32),
            pltpu.VMEM((_PVEC // 2,), jnp.int32),
            pltpu.VMEM((16,), jnp.float32),
            pltpu.VMEM((2 * MAX_F, FT_OUT // 2), jnp.int32),
            pltpu.VMEM((2 * MAX_F, FT_OUT // 2), jnp.int32),
            pltpu.VMEM((2 * MAX_F, FT_OUT // 2), jnp.int32),
            pltpu.VMEM((2 * MAX_F, FT_OUT // 2), jnp.int32),
            pltpu.VMEM((R * 16,), jnp.float32),
            pltpu.VMEM((R,), jnp.float32),
            pltpu.SemaphoreType.DMA,
            pltpu.SemaphoreType.DMA,
            pltpu.SemaphoreType.DMA,
            pltpu.SemaphoreType.DMA,
        ],
    )
    return f(w2, vals_flat, stm, nstm, pvec, obvec)


def kernel(values, stm_indices, nstm_indices, ft_w, ft_b, fft_w, fft_b,
           out_w, out_b):
    w2, pv = _build_w2(ft_w, fft_w, ft_b, fft_b, out_w)
    obvec = jnp.broadcast_to(out_b, (16,)).astype(jnp.float32)
    out = _sc_bag(w2, values.reshape(-1),
                  stm_indices.astype(jnp.int32).reshape(-1),
                  nstm_indices.astype(jnp.int32).reshape(-1),
                  pv.reshape(_PVEC // 2), obvec)
    return out.reshape(B, 1)
